# Initial kernel scaffold; baseline (speedup 1.0000x reference)
import jax
import jax.numpy as jnp
from jax.experimental import pallas as pl


def kernel(queries, keys, alpha):
    raise NotImplementedError("write your pallas kernel here")


# speedup vs baseline: 7.5474x; 7.5474x over previous
import functools

import jax
import jax.numpy as jnp
from jax import lax
from jax.experimental import pallas as pl
from jax.experimental.pallas import tpu as pltpu
from jax.experimental.pallas import tpu_sc as plsc

_K = 16
_D = 16
_DAUG = 24
_BK = 2048
_NB = 49
_KPAD = _NB * _BK
_QB = 128
_NCOL = 1024
_BIG = 1e30
_NW = 32


def _stage1_kernel(qe_ref, ke_ref, w_ref, idx_ref,
                   m1, m2, m3, m4, i1, i2, i3, i4):
    kb = pl.program_id(1)

    @pl.when(kb == 0)
    def _init():
        big = jnp.full((_QB, _NCOL), _BIG, jnp.float32)
        zero = jnp.zeros((_QB, _NCOL), jnp.int32)
        m1[...] = big
        m2[...] = big
        m3[...] = big
        m4[...] = big
        i1[...] = zero
        i2[...] = zero
        i3[...] = zero
        i4[...] = zero

    s = ke_ref[16:17, :] + lax.dot_general(
        qe_ref[...].astype(jnp.bfloat16), ke_ref[...].astype(jnp.bfloat16),
        (((1,), (0,)), ((), ())),
        preferred_element_type=jnp.float32)
    lane = lax.broadcasted_iota(jnp.int32, (_QB, _NCOL), 1)
    base = kb * _BK
    for h in range(_BK // _NCOL):
        sh = s[:, h * _NCOL:(h + 1) * _NCOL]
        nv = lane + (base + h * _NCOL)
        a1 = m1[...]
        a2 = m2[...]
        a3 = m3[...]
        a4 = m4[...]
        b1 = i1[...]
        b2 = i2[...]
        b3 = i3[...]
        b4 = i4[...]
        u1 = sh < a1
        u2 = sh < a2
        u3 = sh < a3
        u4 = sh < a4
        m4[...] = jnp.where(u4, jnp.where(u3, a3, sh), a4)
        i4[...] = jnp.where(u4, jnp.where(u3, b3, nv), b4)
        m3[...] = jnp.where(u3, jnp.where(u2, a2, sh), a3)
        i3[...] = jnp.where(u3, jnp.where(u2, b2, nv), b3)
        m2[...] = jnp.where(u2, jnp.where(u1, a1, sh), a2)
        i2[...] = jnp.where(u2, jnp.where(u1, b1, nv), b2)
        m1[...] = jnp.where(u1, sh, a1)
        i1[...] = jnp.where(u1, nv, b1)

    @pl.when(kb == _NB - 1)
    def _extract():
        a1 = m1[...]
        a2 = m2[...]
        a3 = m3[...]
        a4 = m4[...]
        b1 = i1[...]
        b2 = i2[...]
        b3 = i3[...]
        b4 = i4[...]
        qneg2 = qe_ref[...][:, :_D]
        qsq = 0.25 * jnp.sum(qneg2 * qneg2, axis=1, keepdims=True)
        lanei = lax.broadcasted_iota(jnp.int32, (_QB, _NCOL), 1)
        ec = jnp.zeros((_QB, _NCOL), jnp.int32)
        for r in range(_K):
            h = jnp.where(ec == 0, a1,
                jnp.where(ec == 1, a2,
                jnp.where(ec == 2, a3,
                jnp.where(ec == 3, a4, _BIG))))
            hi = jnp.where(ec == 0, b1,
                 jnp.where(ec == 1, b2,
                 jnp.where(ec == 2, b3, b4)))
            rm = jnp.min(h, axis=1, keepdims=True)
            cstar = jnp.min(jnp.where(h == rm, lanei, _NCOL), axis=1,
                            keepdims=True)
            oh = lanei == cstar
            ir = jnp.max(jnp.where(oh, hi, 0), axis=1, keepdims=True)
            w_ref[:, r:r + 1] = jnp.exp(-0.5 * (rm + qsq))
            idx_ref[:, r:r + 1] = ir
            ec = ec + oh.astype(jnp.int32)


def _stage1(qe, ke):
    nqb = qe.shape[0] // _QB
    return pl.pallas_call(
        _stage1_kernel,
        grid=(nqb, _NB),
        in_specs=[
            pl.BlockSpec((_QB, _DAUG), lambda qb, kb: (qb, 0)),
            pl.BlockSpec((_DAUG, _BK), lambda qb, kb: (0, kb)),
        ],
        out_specs=[
            pl.BlockSpec((_QB, _K), lambda qb, kb: (qb, 0)),
            pl.BlockSpec((_QB, _K), lambda qb, kb: (qb, 0)),
        ],
        out_shape=[
            jax.ShapeDtypeStruct((qe.shape[0], _K), jnp.float32),
            jax.ShapeDtypeStruct((qe.shape[0], _K), jnp.int32),
        ],
        scratch_shapes=(
            [pltpu.VMEM((_QB, _NCOL), jnp.float32)] * 4
            + [pltpu.VMEM((_QB, _NCOL), jnp.int32)] * 4
        ),
        compiler_params=pltpu.CompilerParams(
            dimension_semantics=("parallel", "arbitrary")),
    )(qe, ke)


def _stage2(alpha, idx_flat, w2):
    nq = idx_flat.shape[0] // _K
    nc = alpha.shape[1]
    qpw = nq // _NW
    rpw = qpw * _K
    mesh = plsc.VectorSubcoreMesh(core_axis_name="c", subcore_axis_name="s")

    @functools.partial(
        pl.kernel, mesh=mesh,
        out_type=jax.ShapeDtypeStruct((nq * nc,), jnp.float32),
        compiler_params=pltpu.CompilerParams(use_tc_tiling_on_sc=False),
        scratch_types=[
            pltpu.VMEM((rpw,), jnp.int32),
            pltpu.VMEM((rpw, 16), jnp.float32),
            pltpu.VMEM((rpw, nc), jnp.float32),
            pltpu.VMEM((qpw * nc,), jnp.float32),
            pltpu.SemaphoreType.DMA,
        ],
    )
    def sc_kernel(alpha_hbm, idx_hbm, w2_hbm, out_hbm,
                  idx_v, w2_v, rows_v, out_v, sem):
        wid = lax.axis_index("s") * 2 + lax.axis_index("c")
        rbase = wid * rpw
        pltpu.sync_copy(idx_hbm.at[pl.ds(rbase, rpw)], idx_v)
        pltpu.sync_copy(w2_hbm.at[pl.ds(rbase, rpw)], w2_v)
        pltpu.async_copy(alpha_hbm.at[idx_v], rows_v, sem).wait()

        def qloop(qi, carry):
            def jloop(j, accs):
                r = qi * _K + j
                wj = w2_v[r]
                return tuple(
                    accs[c] + wj * rows_v[r, pl.ds(c * 16, 16)]
                    for c in range(nc // 16))

            accs = lax.fori_loop(
                0, _K, jloop,
                tuple(jnp.zeros((16,), jnp.float32)
                      for _ in range(nc // 16)))
            for c in range(nc // 16):
                out_v[pl.ds(qi * nc + c * 16, 16)] = accs[c]
            return carry

        lax.fori_loop(0, qpw, qloop, 0)
        pltpu.sync_copy(out_v, out_hbm.at[pl.ds(wid * qpw * nc, qpw * nc)])

    return sc_kernel(alpha, idx_flat, w2)


def kernel(queries, keys, alpha):
    nq, d = queries.shape
    nk = keys.shape[0]
    nc = alpha.shape[1]
    f32 = jnp.float32
    qe = jnp.concatenate(
        [-2.0 * queries, jnp.zeros((nq, _DAUG - d), f32)], axis=1)
    ksq = jnp.sum(keys * keys, axis=1)
    kp = jnp.concatenate(
        [keys, ksq[:, None], jnp.zeros((nk, _DAUG - d - 1), f32)], axis=1)
    pad = jnp.zeros((_KPAD - nk, _DAUG), f32).at[:, d].set(_BIG)
    ke = jnp.concatenate([kp, pad], axis=0).T
    w, idx = _stage1(qe, ke)
    w2 = jnp.broadcast_to(w.reshape(-1)[:, None], (nq * _K, 16))
    out_flat = _stage2(alpha, idx.reshape(-1), w2)
    return out_flat.reshape(nq, nc).T


# speedup vs baseline: 8.7766x; 1.1629x over previous
import functools

import jax
import jax.numpy as jnp
from jax import lax
from jax.experimental import pallas as pl
from jax.experimental.pallas import tpu as pltpu
from jax.experimental.pallas import tpu_sc as plsc

_K = 16
_D = 16
_DAUG = 24
_BK = 2048
_NB = 49
_KPAD = _NB * _BK
_QB = 128
_NCOL = 1024
_BIG = 1e30
_NW = 32


def _stage1_kernel(qe_ref, ke_ref, w_ref, idx_ref, m1, m2, m3, m4):
    kb = pl.program_id(1)

    @pl.when(kb == 0)
    def _init():
        big = jnp.full((_QB, _NCOL), _BIG, jnp.float32)
        m1[...] = big
        m2[...] = big
        m3[...] = big
        m4[...] = big

    s = ke_ref[16:17, :] + lax.dot_general(
        qe_ref[...].astype(jnp.bfloat16), ke_ref[...].astype(jnp.bfloat16),
        (((1,), (0,)), ((), ())),
        preferred_element_type=jnp.float32)
    sp = lax.bitcast_convert_type(s, jnp.int32)
    sp = jnp.bitwise_or(jnp.bitwise_and(sp, -128),
                        2 * kb + lax.broadcasted_iota(jnp.int32, s.shape, 1)
                        // _NCOL)
    spf = lax.bitcast_convert_type(sp, jnp.float32)
    for h in range(_BK // _NCOL):
        sh = spf[:, h * _NCOL:(h + 1) * _NCOL]
        a1 = m1[...]
        a2 = m2[...]
        a3 = m3[...]
        a4 = m4[...]
        u1 = sh < a1
        u2 = sh < a2
        u3 = sh < a3
        u4 = sh < a4
        m4[...] = jnp.where(u4, jnp.where(u3, a3, sh), a4)
        m3[...] = jnp.where(u3, jnp.where(u2, a2, sh), a3)
        m2[...] = jnp.where(u2, jnp.where(u1, a1, sh), a2)
        m1[...] = jnp.where(u1, sh, a1)

    @pl.when(kb == _NB - 1)
    def _extract():
        a1 = m1[...]
        a2 = m2[...]
        a3 = m3[...]
        a4 = m4[...]
        qneg2 = qe_ref[...][:, :_D]
        qsq = 0.25 * jnp.sum(qneg2 * qneg2, axis=1, keepdims=True)
        lanei = lax.broadcasted_iota(jnp.int32, (_QB, _NCOL), 1)
        ec = jnp.zeros((_QB, _NCOL), jnp.int32)
        for r in range(_K):
            h = jnp.where(ec == 0, a1,
                jnp.where(ec == 1, a2,
                jnp.where(ec == 2, a3,
                jnp.where(ec == 3, a4, _BIG))))
            rm = jnp.min(h, axis=1, keepdims=True)
            cstar = jnp.min(jnp.where(h == rm, lanei, _NCOL), axis=1,
                            keepdims=True)
            oh = lanei == cstar
            bits = lax.bitcast_convert_type(rm, jnp.int32)
            row = jnp.bitwise_and(bits, 127)
            val = lax.bitcast_convert_type(
                jnp.bitwise_and(bits, -128), jnp.float32)
            w_ref[:, r:r + 1] = jnp.exp(-0.5 * (val + qsq))
            idx_ref[:, r:r + 1] = row * _NCOL + cstar
            ec = ec + oh.astype(jnp.int32)


def _stage1(qe, ke):
    nqb = qe.shape[0] // _QB
    return pl.pallas_call(
        _stage1_kernel,
        grid=(nqb, _NB),
        in_specs=[
            pl.BlockSpec((_QB, _DAUG), lambda qb, kb: (qb, 0)),
            pl.BlockSpec((_DAUG, _BK), lambda qb, kb: (0, kb)),
        ],
        out_specs=[
            pl.BlockSpec((_QB, _K), lambda qb, kb: (qb, 0)),
            pl.BlockSpec((_QB, _K), lambda qb, kb: (qb, 0)),
        ],
        out_shape=[
            jax.ShapeDtypeStruct((qe.shape[0], _K), jnp.float32),
            jax.ShapeDtypeStruct((qe.shape[0], _K), jnp.int32),
        ],
        scratch_shapes=[pltpu.VMEM((_QB, _NCOL), jnp.float32)] * 4,
        compiler_params=pltpu.CompilerParams(
            dimension_semantics=("parallel", "arbitrary")),
    )(qe, ke)


def _stage2(alpha, idx_flat, w2):
    nq = idx_flat.shape[0] // _K
    nc = alpha.shape[1]
    qpw = nq // _NW
    rpw = qpw * _K
    mesh = plsc.VectorSubcoreMesh(core_axis_name="c", subcore_axis_name="s")

    @functools.partial(
        pl.kernel, mesh=mesh,
        out_type=jax.ShapeDtypeStruct((nq * nc,), jnp.float32),
        compiler_params=pltpu.CompilerParams(use_tc_tiling_on_sc=False),
        scratch_types=[
            pltpu.VMEM((rpw,), jnp.int32),
            pltpu.VMEM((rpw, 16), jnp.float32),
            pltpu.VMEM((rpw, nc), jnp.float32),
            pltpu.VMEM((qpw * nc,), jnp.float32),
            pltpu.SemaphoreType.DMA,
        ],
    )
    def sc_kernel(alpha_hbm, idx_hbm, w2_hbm, out_hbm,
                  idx_v, w2_v, rows_v, out_v, sem):
        wid = lax.axis_index("s") * 2 + lax.axis_index("c")
        rbase = wid * rpw
        pltpu.sync_copy(idx_hbm.at[pl.ds(rbase, rpw)], idx_v)
        pltpu.sync_copy(w2_hbm.at[pl.ds(rbase, rpw)], w2_v)
        pltpu.async_copy(alpha_hbm.at[idx_v], rows_v, sem).wait()

        def qloop(qi, carry):
            def jloop(j, accs):
                r = qi * _K + j
                wj = w2_v[r]
                return tuple(
                    accs[c] + wj * rows_v[r, pl.ds(c * 16, 16)]
                    for c in range(nc // 16))

            accs = lax.fori_loop(
                0, _K, jloop,
                tuple(jnp.zeros((16,), jnp.float32)
                      for _ in range(nc // 16)))
            for c in range(nc // 16):
                out_v[pl.ds(qi * nc + c * 16, 16)] = accs[c]
            return carry

        lax.fori_loop(0, qpw, qloop, 0)
        pltpu.sync_copy(out_v, out_hbm.at[pl.ds(wid * qpw * nc, qpw * nc)])

    return sc_kernel(alpha, idx_flat, w2)


def kernel(queries, keys, alpha):
    nq, d = queries.shape
    nk = keys.shape[0]
    nc = alpha.shape[1]
    f32 = jnp.float32
    qe = jnp.concatenate(
        [-2.0 * queries, jnp.zeros((nq, _DAUG - d), f32)], axis=1)
    ksq = jnp.sum(keys * keys, axis=1)
    kp = jnp.concatenate(
        [keys, ksq[:, None], jnp.zeros((nk, _DAUG - d - 1), f32)], axis=1)
    pad = jnp.zeros((_KPAD - nk, _DAUG), f32).at[:, d].set(_BIG)
    ke = jnp.concatenate([kp, pad], axis=0).T
    w, idx = _stage1(qe, ke)
    w2 = jnp.broadcast_to(w.reshape(-1)[:, None], (nq * _K, 16))
    out_flat = _stage2(alpha, idx.reshape(-1), w2)
    return out_flat.reshape(nq, nc).T


# speedup vs baseline: 9.9927x; 1.1386x over previous
import functools

import jax
import jax.numpy as jnp
from jax import lax
from jax.experimental import pallas as pl
from jax.experimental.pallas import tpu as pltpu
from jax.experimental.pallas import tpu_sc as plsc

_K = 16
_D = 16
_DAUG = 24
_BK = 8192
_NB = 13
_KPAD = _NB * _BK
_QB = 128
_NCOL = 1024
_BIG = 1e30
_NW = 32


def _stage1_kernel(qe_ref, ke_ref, w_ref, idx_ref, m1, m2, m3, m4):
    kb = pl.program_id(1)

    @pl.when(kb == 0)
    def _init():
        big = jnp.full((_QB, _NCOL), _BIG, jnp.float32)
        m1[...] = big
        m2[...] = big
        m3[...] = big
        m4[...] = big

    s = ke_ref[16:17, :] + lax.dot_general(
        qe_ref[...].astype(jnp.bfloat16), ke_ref[...].astype(jnp.bfloat16),
        (((1,), (0,)), ((), ())),
        preferred_element_type=jnp.float32)
    sp = lax.bitcast_convert_type(s, jnp.int32)
    nh = _BK // _NCOL
    for h in range(nh):
        sh = lax.bitcast_convert_type(
            jnp.bitwise_or(
                jnp.bitwise_and(sp[:, h * _NCOL:(h + 1) * _NCOL], -128),
                nh * kb + h),
            jnp.float32)
        a1 = m1[...]
        a2 = m2[...]
        a3 = m3[...]
        a4 = m4[...]
        u1 = sh < a1
        u2 = sh < a2
        u3 = sh < a3
        u4 = sh < a4
        m4[...] = jnp.where(u4, jnp.where(u3, a3, sh), a4)
        m3[...] = jnp.where(u3, jnp.where(u2, a2, sh), a3)
        m2[...] = jnp.where(u2, jnp.where(u1, a1, sh), a2)
        m1[...] = jnp.where(u1, sh, a1)

    @pl.when(kb == _NB - 1)
    def _extract():
        a1 = m1[...]
        a2 = m2[...]
        a3 = m3[...]
        a4 = m4[...]
        qneg2 = qe_ref[...][:, :_D]
        qsq = 0.25 * jnp.sum(qneg2 * qneg2, axis=1, keepdims=True)
        lanei = lax.broadcasted_iota(jnp.int32, (_QB, _NCOL), 1)
        ec = jnp.zeros((_QB, _NCOL), jnp.int32)
        for r in range(_K):
            h = jnp.where(ec == 0, a1,
                jnp.where(ec == 1, a2,
                jnp.where(ec == 2, a3,
                jnp.where(ec == 3, a4, _BIG))))
            rm = jnp.min(h, axis=1, keepdims=True)
            cstar = jnp.min(jnp.where(h == rm, lanei, _NCOL), axis=1,
                            keepdims=True)
            oh = lanei == cstar
            bits = lax.bitcast_convert_type(rm, jnp.int32)
            row = jnp.bitwise_and(bits, 127)
            val = lax.bitcast_convert_type(
                jnp.bitwise_and(bits, -128), jnp.float32)
            w_ref[:, r:r + 1] = jnp.exp(-0.5 * (val + qsq))
            idx_ref[:, r:r + 1] = row * _NCOL + cstar
            ec = ec + oh.astype(jnp.int32)


def _stage1(qe, ke):
    nqb = qe.shape[0] // _QB
    return pl.pallas_call(
        _stage1_kernel,
        grid=(nqb, _NB),
        in_specs=[
            pl.BlockSpec((_QB, _DAUG), lambda qb, kb: (qb, 0)),
            pl.BlockSpec((_DAUG, _BK), lambda qb, kb: (0, kb)),
        ],
        out_specs=[
            pl.BlockSpec((_QB, _K), lambda qb, kb: (qb, 0)),
            pl.BlockSpec((_QB, _K), lambda qb, kb: (qb, 0)),
        ],
        out_shape=[
            jax.ShapeDtypeStruct((qe.shape[0], _K), jnp.float32),
            jax.ShapeDtypeStruct((qe.shape[0], _K), jnp.int32),
        ],
        scratch_shapes=[pltpu.VMEM((_QB, _NCOL), jnp.float32)] * 4,
        compiler_params=pltpu.CompilerParams(
            dimension_semantics=("parallel", "arbitrary")),
    )(qe, ke)


def _stage2(alpha, idx_flat, w2):
    nq = idx_flat.shape[0] // _K
    nc = alpha.shape[1]
    qpw = nq // _NW
    rpw = qpw * _K
    mesh = plsc.VectorSubcoreMesh(core_axis_name="c", subcore_axis_name="s")

    @functools.partial(
        pl.kernel, mesh=mesh,
        out_type=jax.ShapeDtypeStruct((nq * nc,), jnp.float32),
        compiler_params=pltpu.CompilerParams(use_tc_tiling_on_sc=False),
        scratch_types=[
            pltpu.VMEM((rpw,), jnp.int32),
            pltpu.VMEM((rpw, 16), jnp.float32),
            pltpu.VMEM((rpw, nc), jnp.float32),
            pltpu.VMEM((qpw * nc,), jnp.float32),
            pltpu.SemaphoreType.DMA,
        ],
    )
    def sc_kernel(alpha_hbm, idx_hbm, w2_hbm, out_hbm,
                  idx_v, w2_v, rows_v, out_v, sem):
        wid = lax.axis_index("s") * 2 + lax.axis_index("c")
        rbase = wid * rpw
        pltpu.sync_copy(idx_hbm.at[pl.ds(rbase, rpw)], idx_v)
        pltpu.sync_copy(w2_hbm.at[pl.ds(rbase, rpw)], w2_v)
        pltpu.async_copy(alpha_hbm.at[idx_v], rows_v, sem).wait()

        def qloop(qi, carry):
            def jloop(j, accs):
                r = qi * _K + j
                wj = w2_v[r]
                return tuple(
                    accs[c] + wj * rows_v[r, pl.ds(c * 16, 16)]
                    for c in range(nc // 16))

            accs = lax.fori_loop(
                0, _K, jloop,
                tuple(jnp.zeros((16,), jnp.float32)
                      for _ in range(nc // 16)))
            for c in range(nc // 16):
                out_v[pl.ds(qi * nc + c * 16, 16)] = accs[c]
            return carry

        lax.fori_loop(0, qpw, qloop, 0)
        pltpu.sync_copy(out_v, out_hbm.at[pl.ds(wid * qpw * nc, qpw * nc)])

    return sc_kernel(alpha, idx_flat, w2)


def kernel(queries, keys, alpha):
    nq, d = queries.shape
    nk = keys.shape[0]
    nc = alpha.shape[1]
    f32 = jnp.float32
    qe = jnp.concatenate(
        [-2.0 * queries, jnp.zeros((nq, _DAUG - d), f32)], axis=1)
    ksq = jnp.sum(keys * keys, axis=1)
    kp = jnp.concatenate(
        [keys, ksq[:, None], jnp.zeros((nk, _DAUG - d - 1), f32)], axis=1)
    pad = jnp.zeros((_KPAD - nk, _DAUG), f32).at[:, d].set(_BIG)
    ke = jnp.concatenate([kp, pad], axis=0).T
    w, idx = _stage1(qe, ke)
    w2 = jnp.broadcast_to(w.reshape(-1)[:, None], (nq * _K, 16))
    out_flat = _stage2(alpha, idx.reshape(-1), w2)
    return out_flat.reshape(nq, nc).T
